# Initial kernel scaffold; baseline (speedup 1.0000x reference)
import jax
import jax.numpy as jnp
from jax.experimental import pallas as pl


def kernel(x, edge_index, W1, b1, W_mu, b_mu, W_logstd, b_logstd):
    raise NotImplementedError("write your pallas kernel here")


# speedup vs baseline: 13.4207x; 13.4207x over previous
import functools

import jax
import jax.numpy as jnp
from jax import lax
from jax.experimental import pallas as pl
from jax.experimental.pallas import tpu as pltpu
from jax.experimental.pallas import tpu_sc as plsc

_D = 128
_NC = 2
_NS = 16
_NW = _NC * _NS
_CK = 128
_NPAD = 10240
_RB = 128
_NBLK = _NPAD // (_NS * _RB)
_R = 512


def _mesh():
    return plsc.VectorSubcoreMesh(core_axis_name="c", subcore_axis_name="s")


@functools.cache
def _make_deg(epad: int):
    ew = epad // _NW
    nchunk = ew // _CK
    zper = _NPAD // _NS

    @functools.partial(
        pl.kernel,
        out_type=jax.ShapeDtypeStruct((_NC, _NPAD), jnp.float32),
        mesh=_mesh(),
        scratch_types=[
            pltpu.VMEM((_CK,), jnp.int32),
            pltpu.VMEM((_CK,), jnp.float32),
            pltpu.VMEM((zper,), jnp.float32),
            pltpu.VMEM_SHARED((_NPAD,), jnp.float32),
        ],
    )
    def deg_kernel(dst_hbm, out_hbm, idx_v, ones_v, zbuf, acc):
        cid = lax.axis_index("c")
        sid = lax.axis_index("s")
        wid = sid * _NC + cid
        ones16 = jnp.full((16,), 1.0, jnp.float32)
        zero16 = jnp.zeros((16,), jnp.float32)
        for j in range(_CK // 16):
            ones_v[pl.ds(j * 16, 16)] = ones16

        def zb(i, c):
            zbuf[pl.ds(i * 16, 16)] = zero16
            return c

        lax.fori_loop(0, zper // 16, zb, 0)
        pltpu.sync_copy(zbuf, acc.at[pl.ds(sid * zper, zper)])
        plsc.subcore_barrier()

        def body(c, carry):
            base = pl.multiple_of(wid * ew + c * _CK, _CK)
            pltpu.sync_copy(dst_hbm.at[pl.ds(base, _CK)], idx_v)
            pltpu.sync_copy(ones_v, acc.at[idx_v], add=True)
            return carry

        lax.fori_loop(0, nchunk, body, 0)
        plsc.subcore_barrier()
        pltpu.sync_copy(acc.at[pl.ds(sid * zper, zper)],
                        out_hbm.at[cid, pl.ds(sid * zper, zper)])

    return deg_kernel


@functools.cache
def _make_agg(epad: int):
    ew = epad // _NW
    nchunk = ew // _CK

    @functools.partial(
        pl.kernel,
        out_type=jax.ShapeDtypeStruct((_NC, _NPAD, _D), jnp.float32),
        mesh=_mesh(),
        scratch_types=[
            pltpu.VMEM((_CK,), jnp.int32),
            pltpu.VMEM((_CK,), jnp.int32),
            pltpu.VMEM((_CK, _D), jnp.float32),
            pltpu.VMEM((_RB, _D), jnp.float32),
            pltpu.VMEM_SHARED((_NPAD, _D), jnp.float32),
            pltpu.SemaphoreType.DMA,
        ],
    )
    def agg_kernel(src_hbm, dst_hbm, u_hbm, out_hbm,
                   sidx, didx, rows, zbuf, acc, sem):
        cid = lax.axis_index("c")
        sid = lax.axis_index("s")
        wid = sid * _NC + cid
        zero16 = jnp.zeros((16,), jnp.float32)

        def zb(i, c):
            for j in range(_D // 16):
                zbuf[i, pl.ds(j * 16, 16)] = zero16
            return c

        lax.fori_loop(0, _RB, zb, 0)
        row0 = sid * (_NBLK * _RB)
        for b in range(_NBLK):
            pltpu.sync_copy(zbuf, acc.at[pl.ds(row0 + b * _RB, _RB), :])
        plsc.subcore_barrier()

        def body(c, carry):
            base = pl.multiple_of(wid * ew + c * _CK, _CK)
            pltpu.sync_copy(src_hbm.at[pl.ds(base, _CK)], sidx)
            pltpu.sync_copy(dst_hbm.at[pl.ds(base, _CK)], didx)
            pltpu.async_copy(u_hbm.at[sidx], rows, sem).wait()
            pltpu.sync_copy(rows, acc.at[didx], add=True)
            return carry

        lax.fori_loop(0, nchunk, body, 0)
        plsc.subcore_barrier()
        for b in range(_NBLK):
            r = row0 + b * _RB
            pltpu.sync_copy(acc.at[pl.ds(r, _RB), :],
                            out_hbm.at[cid, pl.ds(r, _RB), :])

    return agg_kernel


def _dinv_of(dp_ref):
    return lax.rsqrt(dp_ref[0] + dp_ref[1] + 1.0)


def _tc_scale_body(dp_ref, x_ref, u_ref):
    u_ref[...] = x_ref[...] * _dinv_of(dp_ref)


def _tc_hidden_body(dp_ref, s_ref, u1_ref, w_ref, b_ref, u2_ref):
    dinv = _dinv_of(dp_ref)
    ax = (s_ref[0] + s_ref[1] + u1_ref[...]) * dinv
    h = jnp.dot(ax, w_ref[...], preferred_element_type=jnp.float32) + b_ref[...]
    u2_ref[...] = jnp.maximum(h, 0.0) * dinv


def _tc_out_body(dp_ref, s_ref, u2_ref, wmu_ref, bmu_ref, wls_ref, bls_ref,
                 mu_ref, ls_ref):
    dinv = _dinv_of(dp_ref)
    ah = (s_ref[0] + s_ref[1] + u2_ref[...]) * dinv
    mu_ref[...] = jnp.dot(ah, wmu_ref[...],
                          preferred_element_type=jnp.float32) + bmu_ref[...]
    ls_ref[...] = jnp.dot(ah, wls_ref[...],
                          preferred_element_type=jnp.float32) + bls_ref[...]


def _dp_spec():
    return pl.BlockSpec((_NC, _R, 1), lambda i: (0, i, 0))


def _row_spec(d):
    return pl.BlockSpec((_R, d), lambda i: (i, 0))


def _s_spec():
    return pl.BlockSpec((_NC, _R, _D), lambda i: (0, i, 0))


def _full_spec(shape):
    return pl.BlockSpec(shape, lambda i: tuple(0 for _ in shape))


def kernel(x, edge_index, W1, b1, W_mu, b_mu, W_logstd, b_logstd):
    n, d = x.shape
    e = edge_index.shape[1]
    epad = _NW * _CK * (-(-e // (_NW * _CK)))
    npad_e = epad - e
    src = jnp.concatenate(
        [edge_index[0], jnp.zeros((npad_e,), edge_index.dtype)])
    pad_dst = n + (jnp.arange(npad_e, dtype=edge_index.dtype) % (_NPAD - n))
    dst = jnp.concatenate([edge_index[1], pad_dst])

    degp = _make_deg(epad)(dst)
    dp3 = degp[:, :, None]

    grid = (_NPAD // _R,)
    u1 = pl.pallas_call(
        _tc_scale_body,
        grid=grid,
        in_specs=[_dp_spec(), _row_spec(d)],
        out_specs=_row_spec(d),
        out_shape=jax.ShapeDtypeStruct((n, d), jnp.float32),
    )(dp3, x)

    agg = _make_agg(epad)
    s1 = agg(src, dst, u1)

    dhid = W1.shape[1]
    u2 = pl.pallas_call(
        _tc_hidden_body,
        grid=grid,
        in_specs=[_dp_spec(), _s_spec(), _row_spec(d),
                  _full_spec(W1.shape), _full_spec((1, dhid))],
        out_specs=_row_spec(dhid),
        out_shape=jax.ShapeDtypeStruct((n, dhid), jnp.float32),
    )(dp3, s1, u1, W1, b1.reshape(1, -1))

    s2 = agg(src, dst, u2)

    dout = W_mu.shape[1]
    mu, logstd = pl.pallas_call(
        _tc_out_body,
        grid=grid,
        in_specs=[_dp_spec(), _s_spec(), _row_spec(dhid),
                  _full_spec(W_mu.shape), _full_spec((1, dout)),
                  _full_spec(W_logstd.shape), _full_spec((1, dout))],
        out_specs=[_row_spec(dout), _row_spec(dout)],
        out_shape=[jax.ShapeDtypeStruct((n, dout), jnp.float32),
                   jax.ShapeDtypeStruct((n, dout), jnp.float32)],
    )(dp3, s2, u2, W_mu, b_mu.reshape(1, -1), W_logstd, b_logstd.reshape(1, -1))

    return (mu, logstd)
